# baseline (speedup 1.0000x reference)
import jax
import jax.numpy as jnp
from jax import lax
from jax.experimental import pallas as pl
from jax.experimental.pallas import tpu as pltpu

B, N, D, E, T, K = 2, 2048, 768, 8, 16, 2
BT = 512


def _gelu(x):
    return x * 0.5 * (1.0 + lax.erf(x * (2.0 ** -0.5)))


def _moe_body(x_ref, taskoh_ref, tt_ref, wg_ref, bg_ref, we_ref, be_ref,
              wu_ref, bu_ref, out_ref,
              g1_s, g2_s, om_s, a1_s, a2_s, acc1_s, acc2_s):
    e = pl.program_id(2)
    x = x_ref[0]

    @pl.when(e == 0)
    def _gating():
        tvec_all = jnp.dot(taskoh_ref[...], tt_ref[...],
                           preferred_element_type=jnp.float32)
        brow = lax.broadcasted_iota(jnp.int32, (B, D), 0)
        tvec = jnp.sum(jnp.where(brow == pl.program_id(0), tvec_all, 0.0),
                       axis=0, keepdims=True)
        logits = (jnp.dot(x, wg_ref[:D], preferred_element_type=jnp.float32)
                  + jnp.dot(tvec, wg_ref[D:],
                            preferred_element_type=jnp.float32)
                  + bg_ref[...])
        iota = lax.broadcasted_iota(jnp.int32, (BT, E), 1)
        m1 = jnp.max(logits, axis=-1, keepdims=True)
        a1 = jnp.min(jnp.where(logits == m1, iota, E), axis=-1, keepdims=True)
        masked = jnp.where(iota == a1, -jnp.inf, logits)
        m2 = jnp.max(masked, axis=-1, keepdims=True)
        a2 = jnp.min(jnp.where(masked == m2, iota, E), axis=-1, keepdims=True)
        g1 = 1.0 / (1.0 + jnp.exp(m2 - m1))
        g1_s[...] = g1
        g2_s[...] = 1.0 - g1
        om_s[...] = 1.0 - g1
        a1_s[...] = a1
        a2_s[...] = a2

    h_e = (jnp.dot(x, we_ref[0], preferred_element_type=jnp.float32)
           + be_ref[0])
    c1 = jnp.where(a1_s[...] == e, h_e, 0.0)
    c2 = jnp.where(a2_s[...] == e, h_e, 0.0)

    @pl.when(e == 0)
    def _init():
        acc1_s[...] = c1
        acc2_s[...] = c2

    @pl.when(e > 0)
    def _accum():
        acc1_s[...] = acc1_s[...] + c1
        acc2_s[...] = acc2_s[...] + c2

    @pl.when(e == E - 1)
    def _combine():
        h_univ = (jnp.dot(x, wu_ref[...], preferred_element_type=jnp.float32)
                  + bu_ref[...])
        out_ref[0] = (g1_s[...] * _gelu(acc1_s[...])
                      + g2_s[...] * _gelu(acc2_s[...])
                      + om_s[...] * _gelu(h_univ))


@jax.jit
def _moe(tokens, task_onehot, task_table, Wg, bg, We, be, Wu, bu):
    grid = (B, N // BT, E)
    return pl.pallas_call(
        _moe_body,
        grid=grid,
        in_specs=[
            pl.BlockSpec((1, BT, D), lambda b, n, e: (b, n, 0)),
            pl.BlockSpec((B, T), lambda b, n, e: (0, 0)),
            pl.BlockSpec((T, D), lambda b, n, e: (0, 0)),
            pl.BlockSpec((2 * D, E), lambda b, n, e: (0, 0)),
            pl.BlockSpec((1, E), lambda b, n, e: (0, 0)),
            pl.BlockSpec((1, D, D), lambda b, n, e: (e, 0, 0)),
            pl.BlockSpec((1, 1, D), lambda b, n, e: (e, 0, 0)),
            pl.BlockSpec((D, D), lambda b, n, e: (0, 0)),
            pl.BlockSpec((1, D), lambda b, n, e: (0, 0)),
        ],
        out_specs=pl.BlockSpec((1, BT, D), lambda b, n, e: (b, n, 0)),
        out_shape=jax.ShapeDtypeStruct((B, N, D), jnp.float32),
        scratch_shapes=[
            pltpu.VMEM((BT, 1), jnp.float32),
            pltpu.VMEM((BT, 1), jnp.float32),
            pltpu.VMEM((BT, 1), jnp.float32),
            pltpu.VMEM((BT, 1), jnp.int32),
            pltpu.VMEM((BT, 1), jnp.int32),
            pltpu.VMEM((BT, D), jnp.float32),
            pltpu.VMEM((BT, D), jnp.float32),
        ],
        compiler_params=pltpu.CompilerParams(
            dimension_semantics=("parallel", "parallel", "arbitrary")),
    )(tokens, task_onehot, task_table, Wg, bg, We, be, Wu, bu)


def kernel(tokens, task_ids, task_table, Wg, bg, We, be, Wu, bu):
    task_onehot = jax.nn.one_hot(task_ids, T, dtype=jnp.float32)
    return _moe(tokens, task_onehot, task_table, Wg, bg.reshape(1, E),
                We, be.reshape(E, 1, D), Wu, bu.reshape(1, D))


# speedup vs baseline: 1.5468x; 1.5468x over previous
import jax
import jax.numpy as jnp
from jax import lax
from jax.experimental import pallas as pl

B, N, D, E, T, K = 2, 2048, 768, 8, 16, 2
BT = 512


def _gelu(x):
    return x * 0.5 * (1.0 + lax.erf(x * (2.0 ** -0.5)))


def _moe_body(x_ref, taskoh_ref, tt_ref, wg_ref, bg_ref, wcat_ref, bcat_ref,
              wu_ref, bu_ref, out_ref):
    x = x_ref[0]
    tvec_all = jnp.dot(taskoh_ref[...], tt_ref[...],
                       preferred_element_type=jnp.float32)
    brow = lax.broadcasted_iota(jnp.int32, (B, D), 0)
    tvec = jnp.sum(jnp.where(brow == pl.program_id(0), tvec_all, 0.0),
                   axis=0, keepdims=True)
    logits = (jnp.dot(x, wg_ref[:D], preferred_element_type=jnp.float32)
              + jnp.dot(tvec, wg_ref[D:], preferred_element_type=jnp.float32)
              + bg_ref[...])
    iota = lax.broadcasted_iota(jnp.int32, (BT, E), 1)
    m1 = jnp.max(logits, axis=-1, keepdims=True)
    a1 = jnp.min(jnp.where(logits == m1, iota, E), axis=-1, keepdims=True)
    masked = jnp.where(iota == a1, -jnp.inf, logits)
    m2 = jnp.max(masked, axis=-1, keepdims=True)
    a2 = jnp.min(jnp.where(masked == m2, iota, E), axis=-1, keepdims=True)
    g1 = 1.0 / (1.0 + jnp.exp(m2 - m1))
    g2 = 1.0 - g1
    omega = 1.0 - g1

    h_univ = jnp.dot(x, wu_ref[...],
                     preferred_element_type=jnp.float32) + bu_ref[...]
    acc1 = jnp.zeros((BT, D), jnp.float32)
    acc2 = jnp.zeros((BT, D), jnp.float32)
    for e in range(E):
        h_e = (jnp.dot(x, wcat_ref[e],
                       preferred_element_type=jnp.float32) + bcat_ref[e])
        acc1 = acc1 + jnp.where(a1 == e, h_e, 0.0)
        acc2 = acc2 + jnp.where(a2 == e, h_e, 0.0)
    out_ref[0] = g1 * _gelu(acc1) + g2 * _gelu(acc2) + omega * _gelu(h_univ)


@jax.jit
def _moe(tokens, task_onehot, task_table, Wg, bg, Wcat, bcat, Wu, bu):
    grid = (B, N // BT)
    return pl.pallas_call(
        _moe_body,
        grid=grid,
        in_specs=[
            pl.BlockSpec((1, BT, D), lambda b, n: (b, n, 0)),
            pl.BlockSpec((B, T), lambda b, n: (0, 0)),
            pl.BlockSpec((T, D), lambda b, n: (0, 0)),
            pl.BlockSpec((2 * D, E), lambda b, n: (0, 0)),
            pl.BlockSpec((1, E), lambda b, n: (0, 0)),
            pl.BlockSpec((E, D, D), lambda b, n: (0, 0, 0)),
            pl.BlockSpec((E, D), lambda b, n: (0, 0)),
            pl.BlockSpec((D, D), lambda b, n: (0, 0)),
            pl.BlockSpec((1, D), lambda b, n: (0, 0)),
        ],
        out_specs=pl.BlockSpec((1, BT, D), lambda b, n: (b, n, 0)),
        out_shape=jax.ShapeDtypeStruct((B, N, D), jnp.float32),
    )(tokens, task_onehot, task_table, Wg, bg, Wcat, bcat, Wu, bu)


def kernel(tokens, task_ids, task_table, Wg, bg, We, be, Wu, bu):
    task_onehot = jax.nn.one_hot(task_ids, T, dtype=jnp.float32)
    return _moe(tokens, task_onehot, task_table, Wg, bg.reshape(1, E),
                We, be, Wu, bu.reshape(1, D))
